# baseline (speedup 1.0000x reference)
import functools

import jax
import jax.numpy as jnp
from jax import lax
from jax.experimental import pallas as pl
from jax.experimental.pallas import tpu as pltpu
from jax.experimental.pallas import tpu_sc as plsc

_SC_CORES = 2
_SC_SUBCORES = 16
_SC_WORKERS = _SC_CORES * _SC_SUBCORES


def _argmin_body(x_ref, k_ref, o_ref):
    c = pl.program_id(0)
    x = x_ref[0]
    ks = k_ref[0]
    kdim = ks.shape[0]
    dots = lax.dot_general(
        x, ks, (((1,), (1,)), ((), ())),
        precision=lax.Precision.DEFAULT,
        preferred_element_type=jnp.float32)
    x_sq = jnp.sum(x * x, axis=1, keepdims=True)
    ones = jnp.ones((1, ks.shape[1]), jnp.float32)
    k_sq = lax.dot_general(
        ones, ks * ks, (((1,), (1,)), ((), ())),
        precision=lax.Precision.HIGHEST,
        preferred_element_type=jnp.float32)
    dist = (x_sq + k_sq) - 2.0 * dots
    minval = jnp.min(dist, axis=1, keepdims=True)
    kiota = lax.broadcasted_iota(jnp.int32, dist.shape, 1)
    masked = jnp.where(dist == minval, kiota, kdim)
    first = jnp.min(masked, axis=1, keepdims=True)
    o_ref[0] = first + c * kdim


def _finish_body(g_ref, o_ref):
    g = g_ref[...]
    avg = jnp.mean(g, axis=0)
    m = jnp.max(avg, axis=-1, keepdims=True)
    e = jnp.exp(avg - m)
    o_ref[...] = e / jnp.sum(e, axis=-1, keepdims=True)


def _make_sc_gather(rows, vdim):
    b_per_w = rows // _SC_WORKERS
    mesh = plsc.VectorSubcoreMesh(core_axis_name="c", subcore_axis_name="s")

    @functools.partial(
        pl.kernel, mesh=mesh,
        out_type=jax.ShapeDtypeStruct((rows, vdim), jnp.float32),
        scratch_types=[
            pltpu.VMEM((b_per_w,), jnp.int32),
            pltpu.VMEM((b_per_w, vdim), jnp.float32),
            pltpu.SemaphoreType.DMA,
        ],
    )
    def gather_kernel(table_hbm, idx_hbm, out_hbm, idx_v, rows_v, sem):
        wid = lax.axis_index("s") * _SC_CORES + lax.axis_index("c")
        base = wid * b_per_w
        pltpu.sync_copy(idx_hbm.at[pl.ds(base, b_per_w)], idx_v)
        pltpu.async_copy(table_hbm.at[idx_v], rows_v, sem).wait()
        pltpu.sync_copy(rows_v, out_hbm.at[pl.ds(base, b_per_w)])

    return gather_kernel


def kernel(batch, keys, values):
    B, C, D = batch.shape
    K = keys.shape[1]
    V = values.shape[-1]

    bt = jnp.transpose(batch, (1, 0, 2))
    idx = pl.pallas_call(
        _argmin_body,
        grid=(C,),
        in_specs=[
            pl.BlockSpec((1, B, D), lambda c: (c, 0, 0)),
            pl.BlockSpec((1, K, D), lambda c: (c, 0, 0)),
        ],
        out_specs=pl.BlockSpec((1, B, 1), lambda c: (c, 0, 0)),
        out_shape=jax.ShapeDtypeStruct((C, B, 1), jnp.int32),
    )(bt, keys)

    table = values.reshape(C * K, V)
    gathered = _make_sc_gather(C * B, V)(table, idx.reshape(C * B))

    out = pl.pallas_call(
        _finish_body,
        out_shape=jax.ShapeDtypeStruct((B, V), jnp.float32),
    )(gathered.reshape(C, B, V))
    return out


# speedup vs baseline: 1.0157x; 1.0157x over previous
import dataclasses
import functools

import jax
import jax.numpy as jnp
from jax import lax
from jax.experimental import pallas as pl
from jax.experimental.pallas import tpu as pltpu
from jax.experimental.pallas import tpu_sc as plsc

_SC_CORES = 2
_SC_SUBCORES = 16
_SC_WORKERS = _SC_CORES * _SC_SUBCORES


def _argmin_body(x_ref, k_ref, o_ref):
    c = pl.program_id(0)
    x = x_ref[0]
    ks = k_ref[0]
    kdim = ks.shape[0]
    dots = lax.dot_general(
        x, ks, (((1,), (1,)), ((), ())),
        precision=lax.Precision.DEFAULT,
        preferred_element_type=jnp.float32)
    x_sq = jnp.sum(x * x, axis=1, keepdims=True)
    ones = jnp.ones((1, ks.shape[1]), jnp.float32)
    k_sq = lax.dot_general(
        ones, ks * ks, (((1,), (1,)), ((), ())),
        precision=lax.Precision.HIGHEST,
        preferred_element_type=jnp.float32)
    dist = (x_sq + k_sq) - 2.0 * dots
    minval = jnp.min(dist, axis=1, keepdims=True)
    kiota = lax.broadcasted_iota(jnp.int32, dist.shape, 1)
    masked = jnp.where(dist == minval, kiota, kdim)
    first = jnp.min(masked, axis=1, keepdims=True)
    o_ref[0] = first + c * kdim


def _make_sc_gather_reduce(nb, nc, vdim):
    b_per_w = nb // _SC_WORKERS
    rows_per_w = b_per_w * nc
    nchunk = vdim // 16
    mesh = plsc.VectorSubcoreMesh(core_axis_name="c", subcore_axis_name="s")
    cp = pltpu.CompilerParams()
    if "needs_layout_passes" in pltpu.CompilerParams.__dataclass_fields__:
        cp = dataclasses.replace(cp, needs_layout_passes=False)

    @functools.partial(
        pl.kernel, mesh=mesh,
        compiler_params=cp,
        out_type=jax.ShapeDtypeStruct((nb, vdim), jnp.float32),
        scratch_types=[
            pltpu.VMEM((rows_per_w,), jnp.int32),
            pltpu.VMEM((rows_per_w, vdim), jnp.float32),
            pltpu.VMEM((b_per_w, vdim), jnp.float32),
            pltpu.SemaphoreType.DMA,
        ],
    )
    def gather_kernel(table_hbm, idx_hbm, out_hbm, idx_v, rows_v, out_v, sem):
        wid = lax.axis_index("s") * _SC_CORES + lax.axis_index("c")
        pltpu.sync_copy(idx_hbm.at[pl.ds(wid * rows_per_w, rows_per_w)], idx_v)
        pltpu.async_copy(table_hbm.at[idx_v], rows_v, sem).wait()
        inv = jnp.float32(1.0 / nc)
        for bb in range(b_per_w):
            vals = []
            for t in range(nchunk):
                a = rows_v[nc * bb, pl.ds(16 * t, 16)]
                for r in range(1, nc):
                    a = a + rows_v[nc * bb + r, pl.ds(16 * t, 16)]
                vals.append(a * inv)
            m = vals[0]
            for t in range(1, nchunk):
                m = jnp.maximum(m, vals[t])
            mm = jnp.max(m)
            es = [jnp.exp(v - mm) for v in vals]
            sv = es[0]
            for t in range(1, nchunk):
                sv = sv + es[t]
            ss = jnp.sum(sv)
            for t in range(nchunk):
                out_v[bb, pl.ds(16 * t, 16)] = es[t] / ss
        pltpu.sync_copy(out_v, out_hbm.at[pl.ds(wid * b_per_w, b_per_w)])

    return gather_kernel


def kernel(batch, keys, values):
    B, C, D = batch.shape
    K = keys.shape[1]
    V = values.shape[-1]

    bt = jnp.transpose(batch, (1, 0, 2))
    idx = pl.pallas_call(
        _argmin_body,
        grid=(C,),
        in_specs=[
            pl.BlockSpec((1, B, D), lambda c: (c, 0, 0)),
            pl.BlockSpec((1, K, D), lambda c: (c, 0, 0)),
        ],
        out_specs=pl.BlockSpec((1, B, 1), lambda c: (c, 0, 0)),
        out_shape=jax.ShapeDtypeStruct((C, B, 1), jnp.int32),
    )(bt, keys)

    idx_bmajor = idx.reshape(C, B).T.reshape(C * B)
    table = values.reshape(C * K, V)
    return _make_sc_gather_reduce(B, C, V)(table, idx_bmajor)
